# Initial kernel scaffold; baseline (speedup 1.0000x reference)
import jax
import jax.numpy as jnp
from jax import lax
from jax.experimental import pallas as pl
from jax.experimental.pallas import tpu as pltpu
from jax.experimental.pallas import tpu_sc as plsc

NC = 2
NS = 16
CHUNK = 125


def _mesh():
    return plsc.VectorSubcoreMesh(core_axis_name="c", subcore_axis_name="s")


def _deg_kernel(n_nodes, n_edges):
    nchunk = n_edges // (NC * NS * CHUNK)
    rows_per_tile = n_nodes // NS

    def body(dst_hbm, out_hbm, dstv, onesv, zrows, dega, sem):
        c = lax.axis_index("c")
        s = lax.axis_index("s")

        ones16 = jnp.full((16,), 1.0, jnp.float32)
        zero16 = jnp.zeros((16,), jnp.float32)

        def fill_ones(i, _):
            onesv[i] = ones16
            return 0
        lax.fori_loop(0, CHUNK, fill_ones, 0)

        def fill_zero(i, _):
            zrows[i] = zero16
            return 0
        lax.fori_loop(0, rows_per_tile, fill_zero, 0)

        pltpu.sync_copy(zrows, dega.at[pl.ds(s * rows_per_tile, rows_per_tile)])
        plsc.subcore_barrier()

        base_row = (c * NS + s) * nchunk
        pltpu.sync_copy(dst_hbm.at[pl.ds(base_row, nchunk)], dstv)

        def step(j, _):
            pltpu.sync_copy(onesv, dega.at[dstv.at[j]], add=True)
            return 0
        lax.fori_loop(0, nchunk, step, 0)

        plsc.subcore_barrier()
        pltpu.sync_copy(dega.at[pl.ds(s * rows_per_tile, rows_per_tile)],
                        out_hbm.at[c, pl.ds(s * rows_per_tile, rows_per_tile)])

    return pl.kernel(
        body,
        out_type=jax.ShapeDtypeStruct((NC, n_nodes, 16), jnp.float32),
        mesh=_mesh(),
        scratch_types=[
            pltpu.VMEM((nchunk, CHUNK), jnp.int32),
            pltpu.VMEM((CHUNK, 16), jnp.float32),
            pltpu.VMEM((rows_per_tile, 16), jnp.float32),
            pltpu.VMEM_SHARED((n_nodes, 16), jnp.float32),
            pltpu.SemaphoreType.DMA,
        ],
    )


def _agg_kernel(n_nodes, n_edges, d):
    nchunk = n_edges // (NC * NS * CHUNK)
    rows_per_tile = n_nodes // NS

    def body(h_hbm, src_hbm, dst_hbm, out_hbm, idxs, idxd, rows, acc, sem):
        c = lax.axis_index("c")
        s = lax.axis_index("s")

        base_row = (c * NS + s) * nchunk
        pltpu.sync_copy(src_hbm.at[pl.ds(base_row, nchunk)], idxs)
        pltpu.sync_copy(dst_hbm.at[pl.ds(base_row, nchunk)], idxd)

        @pl.when(c == 0)
        def _():
            pltpu.sync_copy(h_hbm.at[pl.ds(s * rows_per_tile, rows_per_tile)],
                            acc.at[pl.ds(s * rows_per_tile, rows_per_tile)])

        @pl.when(c != 0)
        def _():
            zero16 = jnp.zeros((16,), jnp.float32)

            def zfill(i, _):
                for k in range(d // 16):
                    rows[i, pl.ds(k * 16, 16)] = zero16
                return 0
            lax.fori_loop(0, CHUNK, zfill, 0)

            def zcopy(k, _):
                pltpu.sync_copy(
                    rows, acc.at[pl.ds(s * rows_per_tile + k * CHUNK, CHUNK)])
                return 0
            lax.fori_loop(0, rows_per_tile // CHUNK, zcopy, 0)

        plsc.subcore_barrier()

        def step(j, _):
            pltpu.sync_copy(h_hbm.at[idxs.at[j]], rows)
            pltpu.sync_copy(rows, acc.at[idxd.at[j]], add=True)
            return 0
        lax.fori_loop(0, nchunk, step, 0)

        plsc.subcore_barrier()
        pltpu.sync_copy(acc.at[pl.ds(s * rows_per_tile, rows_per_tile)],
                        out_hbm.at[c, pl.ds(s * rows_per_tile, rows_per_tile)])

    return pl.kernel(
        body,
        out_type=jax.ShapeDtypeStruct((NC, n_nodes, d), jnp.float32),
        mesh=_mesh(),
        scratch_types=[
            pltpu.VMEM((nchunk, CHUNK), jnp.int32),
            pltpu.VMEM((nchunk, CHUNK), jnp.int32),
            pltpu.VMEM((CHUNK, d), jnp.float32),
            pltpu.VMEM_SHARED((n_nodes, d), jnp.float32),
            pltpu.SemaphoreType.DMA,
        ],
    )


def _tc1_body(x_ref, w_ref, d0_ref, d1_ref, hp_ref, dinv_ref):
    dinv = lax.rsqrt(d0_ref[...] + d1_ref[...] + 1.0)
    h = jnp.dot(x_ref[...], w_ref[...], preferred_element_type=jnp.float32)
    hp_ref[...] = h * dinv
    dinv_ref[...] = dinv


def _tc2_body(t0_ref, t1_ref, dinv_ref, b1_ref, w2_ref, p1_ref, pb1_ref,
              p2_ref, pb2_ref, z_ref, gp_ref, proj_ref):
    dinv = dinv_ref[...]
    z = dinv * (t0_ref[...] + t1_ref[...]) + b1_ref[...]
    z_ref[...] = z
    h = jnp.maximum(z, 0.0)
    gp_ref[...] = dinv * jnp.dot(h, w2_ref[...],
                                 preferred_element_type=jnp.float32)
    t = jnp.maximum(
        jnp.dot(z, p1_ref[...], preferred_element_type=jnp.float32)
        + pb1_ref[...], 0.0)
    proj_ref[...] = (jnp.dot(t, p2_ref[...], preferred_element_type=jnp.float32)
                     + pb2_ref[...])


def _tc3_body(u0_ref, u1_ref, dinv_ref, b2_ref, out_ref):
    out_ref[...] = (dinv_ref[...] * (u0_ref[...] + u1_ref[...])
                    + b2_ref[...])


def kernel(x, edge_index, W1, b1, W2, b2, P1, pb1, P2, pb2):
    n, _ = x.shape
    d_in = x.shape[1]
    d_hid = W1.shape[1]
    e = edge_index.shape[1]

    src2d = edge_index[0].reshape(e // CHUNK, CHUNK)
    dst2d = edge_index[1].reshape(e // CHUNK, CHUNK)

    deg = _deg_kernel(n, e)(dst2d)
    d0 = deg[0, :, 0:1]
    d1 = deg[1, :, 0:1]

    br = 1000
    grid = (n // br,)
    row = pl.BlockSpec((br, d_hid), lambda i: (i, 0))
    col = pl.BlockSpec((br, 1), lambda i: (i, 0))
    mat = pl.BlockSpec((d_hid, d_hid), lambda i: (0, 0))
    vec = pl.BlockSpec((1, d_hid), lambda i: (0, 0))

    hp, dinv = pl.pallas_call(
        _tc1_body,
        grid=grid,
        in_specs=[row, mat, col, col],
        out_specs=[row, col],
        out_shape=[jax.ShapeDtypeStruct((n, d_hid), jnp.float32),
                   jax.ShapeDtypeStruct((n, 1), jnp.float32)],
    )(x, W1, d0, d1)

    t = _agg_kernel(n, e, d_hid)(hp, src2d, dst2d)

    z, gp, proj = pl.pallas_call(
        _tc2_body,
        grid=grid,
        in_specs=[row, row, col, vec, mat, mat, vec, mat, vec],
        out_specs=[row, row, row],
        out_shape=[jax.ShapeDtypeStruct((n, d_hid), jnp.float32),
                   jax.ShapeDtypeStruct((n, d_in), jnp.float32),
                   jax.ShapeDtypeStruct((n, d_hid), jnp.float32)],
    )(t[0], t[1], dinv, b1.reshape(1, d_hid), W2, P1, pb1.reshape(1, d_hid),
      P2, pb2.reshape(1, d_hid))

    u = _agg_kernel(n, e, d_in)(gp, src2d, dst2d)

    out2 = pl.pallas_call(
        _tc3_body,
        grid=grid,
        in_specs=[row, row, col, vec],
        out_specs=row,
        out_shape=jax.ShapeDtypeStruct((n, d_in), jnp.float32),
    )(u[0], u[1], dinv, b2.reshape(1, d_in))

    return (out2, z, proj)


# speedup vs baseline: 11.8162x; 11.8162x over previous
import jax
import jax.numpy as jnp
from jax import lax
from jax.experimental import pallas as pl
from jax.experimental.pallas import tpu as pltpu
from jax.experimental.pallas import tpu_sc as plsc

NC = 2
NS = 16
CHUNK = 125


def _mesh():
    return plsc.VectorSubcoreMesh(core_axis_name="c", subcore_axis_name="s")


def _agg_kernel(n_nodes, n_edges, d):
    nchunk = n_edges // (NC * NS * CHUNK)
    rpt = n_nodes // NS

    def body(h_hbm, h3d_hbm, src_hbm, dst_hbm, zero_hbm, out_hbm,
             idxs, idxd, rows, acc, sem):
        c = lax.axis_index("c")
        s = lax.axis_index("s")

        base_row = (c * NS + s) * nchunk
        pltpu.sync_copy(src_hbm.at[pl.ds(base_row, nchunk)], idxs)
        pltpu.sync_copy(dst_hbm.at[pl.ds(base_row, nchunk)], idxd)

        @pl.when(c == 0)
        def _():
            pltpu.sync_copy(h3d_hbm.at[s], acc.at[pl.ds(s * rpt, rpt)])

        @pl.when(c != 0)
        def _():
            pltpu.sync_copy(zero_hbm, acc.at[pl.ds(s * rpt, rpt)])

        plsc.subcore_barrier()

        def step(j, _):
            pltpu.sync_copy(h_hbm.at[idxs.at[j]], rows)
            pltpu.sync_copy(rows, acc.at[idxd.at[j]], add=True)
            return 0
        lax.fori_loop(0, nchunk, step, 0)

        plsc.subcore_barrier()
        pltpu.sync_copy(acc.at[pl.ds(s * rpt, rpt)], out_hbm.at[c, s])

    return pl.kernel(
        body,
        out_type=jax.ShapeDtypeStruct((NC, NS, rpt, d), jnp.float32),
        mesh=_mesh(),
        scratch_types=[
            pltpu.VMEM((nchunk, CHUNK), jnp.int32),
            pltpu.VMEM((nchunk, CHUNK), jnp.int32),
            pltpu.VMEM((CHUNK, d), jnp.float32),
            pltpu.VMEM_SHARED((n_nodes, d), jnp.float32),
            pltpu.SemaphoreType.DMA,
        ],
    )


def _tc1_body(x_ref, w_ref, deg_ref, hp_ref, dinv_ref):
    dinv = lax.rsqrt(deg_ref[...] + 1.0)
    h = jnp.dot(x_ref[...], w_ref[...], preferred_element_type=jnp.float32)
    hp_ref[...] = h * dinv
    dinv_ref[...] = dinv


def _tc2_body(t0_ref, t1_ref, dinv_ref, b1_ref, w2_ref, p1_ref, pb1_ref,
              p2_ref, pb2_ref, z_ref, gp_ref, proj_ref):
    dinv = dinv_ref[...]
    z = dinv * (t0_ref[...] + t1_ref[...]) + b1_ref[...]
    z_ref[...] = z
    h = jnp.maximum(z, 0.0)
    gp_ref[...] = dinv * jnp.dot(h, w2_ref[...],
                                 preferred_element_type=jnp.float32)
    t = jnp.maximum(
        jnp.dot(z, p1_ref[...], preferred_element_type=jnp.float32)
        + pb1_ref[...], 0.0)
    proj_ref[...] = (jnp.dot(t, p2_ref[...], preferred_element_type=jnp.float32)
                     + pb2_ref[...])


def _tc3_body(u0_ref, u1_ref, dinv_ref, b2_ref, out_ref):
    out_ref[...] = (dinv_ref[...] * (u0_ref[...] + u1_ref[...])
                    + b2_ref[...])


def kernel(x, edge_index, W1, b1, W2, b2, P1, pb1, P2, pb2):
    n = x.shape[0]
    d_in = x.shape[1]
    d_hid = W1.shape[1]
    e = edge_index.shape[1]
    rpt = n // NS

    src2d = edge_index[0].reshape(e // CHUNK, CHUNK)
    dst2d = edge_index[1].reshape(e // CHUNK, CHUNK)
    zero_d = jnp.zeros((rpt, d_hid), jnp.float32)

    deg = jnp.zeros((n,), jnp.float32).at[edge_index[1]].add(1.0)
    degcol = deg[:, None]

    br = 1000
    grid = (n // br,)
    row = pl.BlockSpec((br, d_hid), lambda i: (i, 0))
    col = pl.BlockSpec((br, 1), lambda i: (i, 0))
    mat = pl.BlockSpec((d_hid, d_hid), lambda i: (0, 0))
    vec = pl.BlockSpec((1, d_hid), lambda i: (0, 0))

    hp, dinv = pl.pallas_call(
        _tc1_body,
        grid=grid,
        in_specs=[row, mat, col],
        out_specs=[row, col],
        out_shape=[jax.ShapeDtypeStruct((n, d_hid), jnp.float32),
                   jax.ShapeDtypeStruct((n, 1), jnp.float32)],
    )(x, W1, degcol)

    t = _agg_kernel(n, e, d_hid)(
        hp, hp.reshape(NS, rpt, d_hid), src2d, dst2d, zero_d)

    z, gp, proj = pl.pallas_call(
        _tc2_body,
        grid=grid,
        in_specs=[row, row, col, vec, mat, mat, vec, mat, vec],
        out_specs=[row, row, row],
        out_shape=[jax.ShapeDtypeStruct((n, d_hid), jnp.float32),
                   jax.ShapeDtypeStruct((n, d_in), jnp.float32),
                   jax.ShapeDtypeStruct((n, d_hid), jnp.float32)],
    )(t.reshape(NC, n, d_hid)[0], t.reshape(NC, n, d_hid)[1], dinv,
      b1.reshape(1, d_hid), W2, P1, pb1.reshape(1, d_hid),
      P2, pb2.reshape(1, d_hid))

    u = _agg_kernel(n, e, d_in)(
        gp, gp.reshape(NS, rpt, d_in), src2d, dst2d, zero_d)

    out2 = pl.pallas_call(
        _tc3_body,
        grid=grid,
        in_specs=[row, row, col, vec],
        out_specs=row,
        out_shape=jax.ShapeDtypeStruct((n, d_in), jnp.float32),
    )(u.reshape(NC, n, d_in)[0], u.reshape(NC, n, d_in)[1], dinv,
      b2.reshape(1, d_in))

    return (out2, z, proj)


# speedup vs baseline: 12.5842x; 1.0650x over previous
import jax
import jax.numpy as jnp
from jax import lax
from jax.experimental import pallas as pl
from jax.experimental.pallas import tpu as pltpu
from jax.experimental.pallas import tpu_sc as plsc

NC = 2
NS = 16
CHUNK = 125


def _mesh():
    return plsc.VectorSubcoreMesh(core_axis_name="c", subcore_axis_name="s")


NBUF = 2


def _agg_kernel(n_nodes, n_edges, d):
    nchunk = n_edges // (NC * NS * CHUNK)
    nhalf = nchunk // 2
    ngroup = nhalf // NBUF
    rpt = n_nodes // NS

    def body(h_hbm, h3d_hbm, src_hbm, dst_hbm, zero_hbm, out_hbm,
             idxs, idxd, r0, r1, acc, g0, g1, s0, s1):
        bufs = [r0, r1]
        gsems = [g0, g1]
        ssems = [s0, s1]
        c = lax.axis_index("c")
        s = lax.axis_index("s")

        base_row = (c * NS + s) * nchunk

        @pl.when(c == 0)
        def _():
            pltpu.sync_copy(h3d_hbm.at[s], acc.at[pl.ds(s * rpt, rpt)])

        @pl.when(c != 0)
        def _():
            pltpu.sync_copy(zero_hbm, acc.at[pl.ds(s * rpt, rpt)])

        plsc.subcore_barrier()

        for half in range(2):
            pltpu.sync_copy(
                src_hbm.at[pl.ds(base_row + half * nhalf, nhalf)], idxs)
            pltpu.sync_copy(
                dst_hbm.at[pl.ds(base_row + half * nhalf, nhalf)], idxd)

            for b in range(NBUF):
                pltpu.async_copy(h_hbm.at[idxs.at[b]], bufs[b], gsems[b])

            def group(g, _):
                j0 = g * NBUF
                for b in range(NBUF):
                    pltpu.make_async_copy(
                        h_hbm.at[idxs.at[j0 + b]], bufs[b], gsems[b]).wait()
                    pltpu.async_copy(
                        bufs[b], acc.at[idxd.at[j0 + b]], ssems[b], add=True)
                for b in range(NBUF):
                    pltpu.make_async_copy(
                        bufs[b], acc.at[idxd.at[j0 + b]], ssems[b]).wait()

                    @pl.when(g < ngroup - 1)
                    def _(b=b):
                        pltpu.async_copy(
                            h_hbm.at[idxs.at[j0 + NBUF + b]], bufs[b],
                            gsems[b])
                return 0
            lax.fori_loop(0, ngroup, group, 0)

        plsc.subcore_barrier()
        pltpu.sync_copy(acc.at[pl.ds(s * rpt, rpt)], out_hbm.at[c, s])

    return pl.kernel(
        body,
        out_type=jax.ShapeDtypeStruct((NC, NS, rpt, d), jnp.float32),
        mesh=_mesh(),
        scratch_types=[
            pltpu.VMEM((nhalf, CHUNK), jnp.int32),
            pltpu.VMEM((nhalf, CHUNK), jnp.int32),
            pltpu.VMEM((CHUNK, d), jnp.float32),
            pltpu.VMEM((CHUNK, d), jnp.float32),
            pltpu.VMEM_SHARED((n_nodes, d), jnp.float32),
            pltpu.SemaphoreType.DMA,
            pltpu.SemaphoreType.DMA,
            pltpu.SemaphoreType.DMA,
            pltpu.SemaphoreType.DMA,
        ],
    )


def _tc1a_body(x_ref, w_ref, h_ref):
    h_ref[...] = jnp.dot(x_ref[...], w_ref[...],
                         preferred_element_type=jnp.float32)


def _tc1b_body(h_ref, deg_ref, hp_ref, dinv_ref):
    dinv = lax.rsqrt(deg_ref[...] + 1.0)
    hp_ref[...] = h_ref[...] * dinv
    dinv_ref[...] = dinv


def _tc2_body(t0_ref, t1_ref, dinv_ref, b1_ref, w2_ref, p1_ref, pb1_ref,
              p2_ref, pb2_ref, z_ref, gp_ref, proj_ref):
    dinv = dinv_ref[...]
    z = dinv * (t0_ref[...] + t1_ref[...]) + b1_ref[...]
    z_ref[...] = z
    h = jnp.maximum(z, 0.0)
    gp_ref[...] = dinv * jnp.dot(h, w2_ref[...],
                                 preferred_element_type=jnp.float32)
    t = jnp.maximum(
        jnp.dot(z, p1_ref[...], preferred_element_type=jnp.float32)
        + pb1_ref[...], 0.0)
    proj_ref[...] = (jnp.dot(t, p2_ref[...], preferred_element_type=jnp.float32)
                     + pb2_ref[...])


def _tc3_body(u0_ref, u1_ref, dinv_ref, b2_ref, out_ref):
    out_ref[...] = (dinv_ref[...] * (u0_ref[...] + u1_ref[...])
                    + b2_ref[...])


def kernel(x, edge_index, W1, b1, W2, b2, P1, pb1, P2, pb2):
    n = x.shape[0]
    d_in = x.shape[1]
    d_hid = W1.shape[1]
    e = edge_index.shape[1]
    rpt = n // NS

    src2d = edge_index[0].reshape(e // CHUNK, CHUNK)
    dst2d = edge_index[1].reshape(e // CHUNK, CHUNK)
    zero_d = jnp.zeros((rpt, d_hid), jnp.float32)

    deg = jnp.zeros((n,), jnp.float32).at[edge_index[1]].add(1.0)
    degcol = deg[:, None]

    br = 1000
    grid = (n // br,)
    row = pl.BlockSpec((br, d_hid), lambda i: (i, 0))
    col = pl.BlockSpec((br, 1), lambda i: (i, 0))
    mat = pl.BlockSpec((d_hid, d_hid), lambda i: (0, 0))
    vec = pl.BlockSpec((1, d_hid), lambda i: (0, 0))

    h1 = pl.pallas_call(
        _tc1a_body,
        grid=grid,
        in_specs=[row, mat],
        out_specs=row,
        out_shape=jax.ShapeDtypeStruct((n, d_hid), jnp.float32),
    )(x, W1)

    hp, dinv = pl.pallas_call(
        _tc1b_body,
        grid=grid,
        in_specs=[row, col],
        out_specs=[row, col],
        out_shape=[jax.ShapeDtypeStruct((n, d_hid), jnp.float32),
                   jax.ShapeDtypeStruct((n, 1), jnp.float32)],
    )(h1, degcol)

    t = _agg_kernel(n, e, d_hid)(
        hp, hp.reshape(NS, rpt, d_hid), src2d, dst2d, zero_d)

    z, gp, proj = pl.pallas_call(
        _tc2_body,
        grid=grid,
        in_specs=[row, row, col, vec, mat, mat, vec, mat, vec],
        out_specs=[row, row, row],
        out_shape=[jax.ShapeDtypeStruct((n, d_hid), jnp.float32),
                   jax.ShapeDtypeStruct((n, d_in), jnp.float32),
                   jax.ShapeDtypeStruct((n, d_hid), jnp.float32)],
    )(t.reshape(NC, n, d_hid)[0], t.reshape(NC, n, d_hid)[1], dinv,
      b1.reshape(1, d_hid), W2, P1, pb1.reshape(1, d_hid),
      P2, pb2.reshape(1, d_hid))

    u = _agg_kernel(n, e, d_in)(
        gp, gp.reshape(NS, rpt, d_in), src2d, dst2d, zero_d)

    out2 = pl.pallas_call(
        _tc3_body,
        grid=grid,
        in_specs=[row, row, col, vec],
        out_specs=row,
        out_shape=jax.ShapeDtypeStruct((n, d_in), jnp.float32),
    )(u.reshape(NC, n, d_in)[0], u.reshape(NC, n, d_in)[1], dinv,
      b2.reshape(1, d_in))

    return (out2, z, proj)


# speedup vs baseline: 13.1326x; 1.0436x over previous
import jax
import jax.numpy as jnp
from jax import lax
from jax.experimental import pallas as pl
from jax.experimental.pallas import tpu as pltpu
from jax.experimental.pallas import tpu_sc as plsc

NC = 2
NS = 16
CHUNK = 50


def _mesh():
    return plsc.VectorSubcoreMesh(core_axis_name="c", subcore_axis_name="s")


NBUF = 4


def _agg_kernel(n_nodes, n_edges, d):
    nchunk = n_edges // (NC * NS * CHUNK)
    nseg = 5
    nhalf = nchunk // nseg
    ngroup = nhalf // NBUF
    rpt = n_nodes // NS

    def body(h_hbm, h3d_hbm, src_hbm, dst_hbm, zero_hbm, out_hbm,
             idxs, idxd, r0, r1, r2, r3, acc,
             g0, g1, g2, g3, s0, s1, s2, s3):
        bufs = [r0, r1, r2, r3]
        gsems = [g0, g1, g2, g3]
        ssems = [s0, s1, s2, s3]
        c = lax.axis_index("c")
        s = lax.axis_index("s")

        base_row = (c * NS + s) * nchunk

        @pl.when(c == 0)
        def _():
            pltpu.sync_copy(h3d_hbm.at[s], acc.at[pl.ds(s * rpt, rpt)])

        @pl.when(c != 0)
        def _():
            pltpu.sync_copy(zero_hbm, acc.at[pl.ds(s * rpt, rpt)])

        plsc.subcore_barrier()

        for half in range(nseg):
            pltpu.sync_copy(
                src_hbm.at[pl.ds(base_row + half * nhalf, nhalf)], idxs)
            pltpu.sync_copy(
                dst_hbm.at[pl.ds(base_row + half * nhalf, nhalf)], idxd)

            for b in range(NBUF):
                pltpu.async_copy(h_hbm.at[idxs.at[b]], bufs[b], gsems[b])

            def group(g, _):
                j0 = g * NBUF
                for b in range(NBUF):
                    pltpu.make_async_copy(
                        h_hbm.at[idxs.at[j0 + b]], bufs[b], gsems[b]).wait()
                    pltpu.async_copy(
                        bufs[b], acc.at[idxd.at[j0 + b]], ssems[b], add=True)
                for b in range(NBUF):
                    pltpu.make_async_copy(
                        bufs[b], acc.at[idxd.at[j0 + b]], ssems[b]).wait()

                    @pl.when(g < ngroup - 1)
                    def _(b=b):
                        pltpu.async_copy(
                            h_hbm.at[idxs.at[j0 + NBUF + b]], bufs[b],
                            gsems[b])
                return 0
            lax.fori_loop(0, ngroup, group, 0)

        plsc.subcore_barrier()
        pltpu.sync_copy(acc.at[pl.ds(s * rpt, rpt)], out_hbm.at[c, s])

    return pl.kernel(
        body,
        out_type=jax.ShapeDtypeStruct((NC, NS, rpt, d), jnp.float32),
        mesh=_mesh(),
        scratch_types=[
            pltpu.VMEM((nhalf, CHUNK), jnp.int32),
            pltpu.VMEM((nhalf, CHUNK), jnp.int32),
            pltpu.VMEM((CHUNK, d), jnp.float32),
            pltpu.VMEM((CHUNK, d), jnp.float32),
            pltpu.VMEM((CHUNK, d), jnp.float32),
            pltpu.VMEM((CHUNK, d), jnp.float32),
            pltpu.VMEM_SHARED((n_nodes, d), jnp.float32),
            pltpu.SemaphoreType.DMA,
            pltpu.SemaphoreType.DMA,
            pltpu.SemaphoreType.DMA,
            pltpu.SemaphoreType.DMA,
            pltpu.SemaphoreType.DMA,
            pltpu.SemaphoreType.DMA,
            pltpu.SemaphoreType.DMA,
            pltpu.SemaphoreType.DMA,
        ],
    )


def _tc1a_body(x_ref, w_ref, h_ref):
    h_ref[...] = jnp.dot(x_ref[...], w_ref[...],
                         preferred_element_type=jnp.float32)


def _tc1b_body(h_ref, deg_ref, hp_ref, dinv_ref):
    dinv = lax.rsqrt(deg_ref[...] + 1.0)
    hp_ref[...] = h_ref[...] * dinv
    dinv_ref[...] = dinv


def _tc2_body(t0_ref, t1_ref, dinv_ref, b1_ref, w2_ref, p1_ref, pb1_ref,
              p2_ref, pb2_ref, z_ref, gp_ref, proj_ref):
    dinv = dinv_ref[...]
    z = dinv * (t0_ref[...] + t1_ref[...]) + b1_ref[...]
    z_ref[...] = z
    h = jnp.maximum(z, 0.0)
    gp_ref[...] = dinv * jnp.dot(h, w2_ref[...],
                                 preferred_element_type=jnp.float32)
    t = jnp.maximum(
        jnp.dot(z, p1_ref[...], preferred_element_type=jnp.float32)
        + pb1_ref[...], 0.0)
    proj_ref[...] = (jnp.dot(t, p2_ref[...], preferred_element_type=jnp.float32)
                     + pb2_ref[...])


def _tc3_body(u0_ref, u1_ref, dinv_ref, b2_ref, out_ref):
    out_ref[...] = (dinv_ref[...] * (u0_ref[...] + u1_ref[...])
                    + b2_ref[...])


def kernel(x, edge_index, W1, b1, W2, b2, P1, pb1, P2, pb2):
    n = x.shape[0]
    d_in = x.shape[1]
    d_hid = W1.shape[1]
    e = edge_index.shape[1]
    rpt = n // NS

    src2d = edge_index[0].reshape(e // CHUNK, CHUNK)
    dst2d = edge_index[1].reshape(e // CHUNK, CHUNK)
    zero_d = jnp.zeros((rpt, d_hid), jnp.float32)

    deg = jnp.zeros((n,), jnp.float32).at[edge_index[1]].add(1.0)
    degcol = deg[:, None]

    br = 1000
    grid = (n // br,)
    row = pl.BlockSpec((br, d_hid), lambda i: (i, 0))
    col = pl.BlockSpec((br, 1), lambda i: (i, 0))
    mat = pl.BlockSpec((d_hid, d_hid), lambda i: (0, 0))
    vec = pl.BlockSpec((1, d_hid), lambda i: (0, 0))

    h1 = pl.pallas_call(
        _tc1a_body,
        grid=grid,
        in_specs=[row, mat],
        out_specs=row,
        out_shape=jax.ShapeDtypeStruct((n, d_hid), jnp.float32),
    )(x, W1)

    hp, dinv = pl.pallas_call(
        _tc1b_body,
        grid=grid,
        in_specs=[row, col],
        out_specs=[row, col],
        out_shape=[jax.ShapeDtypeStruct((n, d_hid), jnp.float32),
                   jax.ShapeDtypeStruct((n, 1), jnp.float32)],
    )(h1, degcol)

    t = _agg_kernel(n, e, d_hid)(
        hp, hp.reshape(NS, rpt, d_hid), src2d, dst2d, zero_d)

    z, gp, proj = pl.pallas_call(
        _tc2_body,
        grid=grid,
        in_specs=[row, row, col, vec, mat, mat, vec, mat, vec],
        out_specs=[row, row, row],
        out_shape=[jax.ShapeDtypeStruct((n, d_hid), jnp.float32),
                   jax.ShapeDtypeStruct((n, d_in), jnp.float32),
                   jax.ShapeDtypeStruct((n, d_hid), jnp.float32)],
    )(t.reshape(NC, n, d_hid)[0], t.reshape(NC, n, d_hid)[1], dinv,
      b1.reshape(1, d_hid), W2, P1, pb1.reshape(1, d_hid),
      P2, pb2.reshape(1, d_hid))

    u = _agg_kernel(n, e, d_in)(
        gp, gp.reshape(NS, rpt, d_in), src2d, dst2d, zero_d)

    out2 = pl.pallas_call(
        _tc3_body,
        grid=grid,
        in_specs=[row, row, col, vec],
        out_specs=row,
        out_shape=jax.ShapeDtypeStruct((n, d_in), jnp.float32),
    )(u.reshape(NC, n, d_in)[0], u.reshape(NC, n, d_in)[1], dinv,
      b2.reshape(1, d_in))

    return (out2, z, proj)
